# baseline (speedup 1.0000x reference)
import functools

import jax
import jax.numpy as jnp
from jax import lax
from jax.experimental import pallas as pl
from jax.experimental.pallas import tpu as pltpu
from jax.experimental.pallas import tpu_sc as plsc

S = 16384
D = 32
N = 1000

_info = plsc.get_sparse_core_info()
_NC, _NS = _info.num_cores, _info.num_subcores
_NW = _NC * _NS
_BPW = S // _NW
_CHUNK = 128
_NCH = _BPW // _CHUNK

_sc_mesh = plsc.VectorSubcoreMesh(core_axis_name="c", subcore_axis_name="s")


@functools.partial(
    pl.kernel,
    mesh=_sc_mesh,
    out_type=jax.ShapeDtypeStruct((S, D), jnp.float32),
    scratch_types=[
        pltpu.VMEM((_NCH, _CHUNK), jnp.int32),
        pltpu.VMEM((_BPW, D), jnp.float32),
        pltpu.SemaphoreType.DMA,
    ],
    compiler_params=pltpu.CompilerParams(use_tc_tiling_on_sc=False),
)
def _sc_gather(idx_hbm, table_hbm, out_hbm, idx_v, rows_v, sem):
    wid = lax.axis_index("s") * _NC + lax.axis_index("c")
    base = wid * _BPW
    pltpu.sync_copy(idx_hbm.at[pl.ds(wid * _NCH, _NCH)], idx_v)
    copies = [
        pltpu.async_copy(
            table_hbm.at[idx_v.at[j]],
            rows_v.at[pl.ds(j * _CHUNK, _CHUNK)],
            sem,
        )
        for j in range(_NCH)
    ]
    for c in copies:
        c.wait()
    pltpu.sync_copy(rows_v, out_hbm.at[pl.ds(base, _BPW)])


_BS = 1024


def _tc_body(theta_ref, alpha_ref, out_ref):
    u = lax.dot_general(
        theta_ref[...], alpha_ref[...],
        (((1,), (1,)), ((), ())),
        preferred_element_type=jnp.float32,
    )
    m = jnp.max(u, axis=1, keepdims=True)
    e = jnp.exp(u - m)
    s = jnp.sum(e, axis=1, keepdims=True)
    out_ref[...] = (u - m) - jnp.log(s)


_tc_call = pl.pallas_call(
    _tc_body,
    grid=(S // _BS,),
    in_specs=[
        pl.BlockSpec((_BS, D), lambda i: (i, 0)),
        pl.BlockSpec((N, D), lambda i: (0, 0)),
    ],
    out_specs=pl.BlockSpec((_BS, N), lambda i: (i, 0)),
    out_shape=jax.ShapeDtypeStruct((S, N), jnp.float32),
    compiler_params=pltpu.CompilerParams(
        dimension_semantics=("arbitrary",),
    ),
)


def kernel(user_index, theta_user, alpha_item):
    idx = user_index.astype(jnp.int32).reshape(S // _CHUNK, _CHUNK)
    gathered = _sc_gather(idx, theta_user)
    return _tc_call(gathered, alpha_item)


# speedup vs baseline: 1.1015x; 1.1015x over previous
import functools

import jax
import jax.numpy as jnp
from jax import lax
from jax.experimental import pallas as pl
from jax.experimental.pallas import tpu as pltpu
from jax.experimental.pallas import tpu_sc as plsc

S = 16384
D = 32
N = 1000

_info = plsc.get_sparse_core_info()
_NC, _NS = _info.num_cores, _info.num_subcores
_NW = _NC * _NS
_BPW = S // _NW
_CHUNK = 128
_NCH = _BPW // _CHUNK

_sc_mesh = plsc.VectorSubcoreMesh(core_axis_name="c", subcore_axis_name="s")


@functools.partial(
    pl.kernel,
    mesh=_sc_mesh,
    out_type=jax.ShapeDtypeStruct((S, D), jnp.float32),
    scratch_types=[
        pltpu.VMEM((_NCH, _CHUNK), jnp.int32),
        pltpu.VMEM((_BPW, D), jnp.float32),
        pltpu.SemaphoreType.DMA,
    ],
    compiler_params=pltpu.CompilerParams(use_tc_tiling_on_sc=False),
)
def _sc_gather(idx_hbm, table_hbm, out_hbm, idx_v, rows_v, sem):
    wid = lax.axis_index("s") * _NC + lax.axis_index("c")
    base = wid * _BPW
    pltpu.sync_copy(idx_hbm.at[pl.ds(wid * _NCH, _NCH)], idx_v)
    copies = [
        pltpu.async_copy(
            table_hbm.at[idx_v.at[j]],
            rows_v.at[pl.ds(j * _CHUNK, _CHUNK)],
            sem,
        )
        for j in range(_NCH)
    ]
    for c in copies:
        c.wait()
    pltpu.sync_copy(rows_v, out_hbm.at[pl.ds(base, _BPW)])


_BS = 1024


def _tc_body(alpha_ref, g_ref, out_ref):
    u = lax.dot_general(
        alpha_ref[...], g_ref[...],
        (((0,), (1,)), ((), ())),
        preferred_element_type=jnp.float32,
    )
    m = jnp.max(u, axis=0, keepdims=True)
    e = jnp.exp(u - m)
    s = jnp.sum(e, axis=0, keepdims=True)
    out_ref[...] = (u - m) - jnp.log(s)


_tc_call = pl.pallas_call(
    _tc_body,
    grid=(S // _BS,),
    in_specs=[
        pl.BlockSpec((D, N), lambda i: (0, 0)),
        pl.BlockSpec((_BS, D), lambda i: (i, 0)),
    ],
    out_specs=pl.BlockSpec((N, _BS), lambda i: (0, i)),
    out_shape=jax.ShapeDtypeStruct((N, S), jnp.float32),
    compiler_params=pltpu.CompilerParams(
        dimension_semantics=("arbitrary",),
    ),
)


def kernel(user_index, theta_user, alpha_item):
    idx = user_index.astype(jnp.int32).reshape(S // _CHUNK, _CHUNK)
    gathered = _sc_gather(idx, theta_user)
    out_t = _tc_call(alpha_item.T, gathered)
    return out_t.T
